# baseline (speedup 1.0000x reference)
import functools

import jax
import jax.numpy as jnp
from jax import lax
from jax.experimental import pallas as pl
from jax.experimental.pallas import tpu as pltpu, tpu_sc as plsc

N_NODES = 10000
N_EDGES = 160000
N_GRAPHS = 16
D_IN = 4
D_EDGE = 5
HID = 32

NC = 2
NS = 16
NW = NC * NS
CHUNK = 128
CHUNKS_PER_W = 40
EP = NW * CHUNKS_PER_W * CHUNK
E_PER_W = CHUNKS_PER_W * CHUNK
MSGW = 128
NP_ROWS = 10240
ROWS_PER_TILE = NP_ROWS // NS


def _leaky(v):
    return jnp.where(v >= 0, v, 0.1 * v)


_MESH = plsc.VectorSubcoreMesh(core_axis_name="c", subcore_axis_name="s")


LANES = 128


@functools.partial(
    pl.kernel,
    out_type=jax.ShapeDtypeStruct((EP, LANES), jnp.float32),
    mesh=_MESH,
    scratch_types=[
        pltpu.VMEM((CHUNK,), jnp.int32),
        pltpu.VMEM((CHUNK, LANES), jnp.float32),
        pltpu.SemaphoreType.DMA,
    ],
)
def _sc_gather(h_hbm, src_hbm, u_hbm, idx_v, rows_v, sem):
    wid = lax.axis_index("s") * NC + lax.axis_index("c")
    base = wid * E_PER_W

    def body(c, _):
        off = base + c * CHUNK
        pltpu.sync_copy(src_hbm.at[pl.ds(off, CHUNK)], idx_v)
        pltpu.async_copy(h_hbm.at[idx_v], rows_v, sem).wait()
        pltpu.sync_copy(rows_v, u_hbm.at[pl.ds(off, CHUNK)])
        return _

    lax.fori_loop(0, CHUNKS_PER_W, body, None)


@functools.partial(
    pl.kernel,
    out_type=jax.ShapeDtypeStruct((NC, NP_ROWS, MSGW), jnp.float32),
    mesh=_MESH,
    scratch_types=[
        pltpu.VMEM((CHUNK,), jnp.int32),
        pltpu.VMEM((CHUNK, MSGW), jnp.float32),
        pltpu.VMEM_SHARED((NP_ROWS, MSGW), jnp.float32),
    ],
)
def _sc_scatter(msg_hbm, dst_hbm, zero_hbm, acc_hbm, idx_v, msg_v, shared):
    cid = lax.axis_index("c")
    sid = lax.axis_index("s")
    wid = sid * NC + cid
    base = wid * E_PER_W
    row0 = sid * ROWS_PER_TILE

    pltpu.sync_copy(zero_hbm, shared.at[pl.ds(row0, ROWS_PER_TILE)])
    plsc.subcore_barrier()

    def body(c, _):
        off = base + c * CHUNK
        pltpu.sync_copy(dst_hbm.at[pl.ds(off, CHUNK)], idx_v)
        pltpu.sync_copy(msg_hbm.at[pl.ds(off, CHUNK)], msg_v)
        pltpu.sync_copy(msg_v, shared.at[idx_v], add=True)
        return _

    lax.fori_loop(0, CHUNKS_PER_W, body, None)
    plsc.subcore_barrier()
    pltpu.sync_copy(shared.at[pl.ds(row0, ROWS_PER_TILE)],
                    acc_hbm.at[cid, pl.ds(row0, ROWS_PER_TILE)])


def _tc_prep(x, Wp, bp):
    def body(x_ref, w_ref, b_ref, o_ref):
        h = _leaky(
            jnp.dot(x_ref[...], w_ref[...], preferred_element_type=jnp.float32)
            + b_ref[...])
        o_ref[...] = jnp.concatenate(
            [h, jnp.zeros((N_NODES, LANES - HID), jnp.float32)], axis=1)
    return pl.pallas_call(
        body, out_shape=jax.ShapeDtypeStruct((N_NODES, LANES), jnp.float32),
    )(x, Wp, bp)


def _tc_edge_mlp(ea_p, W1, b1, W2, b2):
    BLK = 2048
    grid = EP // BLK

    def body(ea_ref, w1_ref, b1_ref, w2_ref, b2_ref, o_ref):
        e1 = _leaky(jnp.dot(ea_ref[...], w1_ref[...],
                            preferred_element_type=jnp.float32) + b1_ref[...])
        o_ref[...] = _leaky(jnp.dot(e1, w2_ref[...],
                                    preferred_element_type=jnp.float32) + b2_ref[...])

    return pl.pallas_call(
        body,
        grid=(grid,),
        in_specs=[
            pl.BlockSpec((BLK, D_EDGE), lambda i: (i, 0)),
            pl.BlockSpec((D_EDGE, HID), lambda i: (0, 0)),
            pl.BlockSpec((1, HID), lambda i: (0, 0)),
            pl.BlockSpec((HID, HID), lambda i: (0, 0)),
            pl.BlockSpec((1, HID), lambda i: (0, 0)),
        ],
        out_specs=pl.BlockSpec((BLK, HID), lambda i: (i, 0)),
        out_shape=jax.ShapeDtypeStruct((EP, HID), jnp.float32),
    )(ea_p, W1, b1, W2, b2)


def _tc_message(e2, u, W3r, B3):
    BLK = 512
    grid = EP // BLK

    def body(e2_ref, u_ref, w3_ref, b3_ref, o_ref):
        e2b = e2_ref[...]
        ub = u_ref[:, :HID]
        z = jnp.concatenate([e2b[:, k:k + 1] * ub for k in range(HID)], axis=1)
        msg = (jnp.dot(z, w3_ref[...], preferred_element_type=jnp.float32)
               + jnp.dot(ub, b3_ref[...], preferred_element_type=jnp.float32))
        rows = (pl.program_id(0) * BLK
                + lax.broadcasted_iota(jnp.int32, (BLK, 1), 0))
        maskf = (rows < N_EDGES).astype(jnp.float32)
        o_ref[...] = jnp.concatenate(
            [msg * maskf, maskf, jnp.zeros((BLK, MSGW - HID - 1), jnp.float32)],
            axis=1)

    return pl.pallas_call(
        body,
        grid=(grid,),
        in_specs=[
            pl.BlockSpec((BLK, HID), lambda i: (i, 0)),
            pl.BlockSpec((BLK, LANES), lambda i: (i, 0)),
            pl.BlockSpec((HID * HID, HID), lambda i: (0, 0)),
            pl.BlockSpec((HID, HID), lambda i: (0, 0)),
        ],
        out_specs=pl.BlockSpec((BLK, MSGW), lambda i: (i, 0)),
        out_shape=jax.ShapeDtypeStruct((EP, MSGW), jnp.float32),
    )(e2, u, W3r, B3)


def _tc_update(acc2, h128, root, root_b, ln_g, ln_b):
    def body(a_ref, h_ref, r_ref, rb_ref, g_ref, b_ref, o_ref):
        s = a_ref[0] + a_ref[1]
        deg = jnp.maximum(s[:, HID:HID + 1], 1.0)
        hb = h_ref[:, :HID]
        out = (s[:, :HID] / deg
               + jnp.dot(hb, r_ref[...], preferred_element_type=jnp.float32)
               + rb_ref[...])
        mu = jnp.mean(out, axis=-1, keepdims=True)
        var = jnp.mean((out - mu) ** 2, axis=-1, keepdims=True)
        out = (out - mu) / jnp.sqrt(var + 1e-5) * g_ref[...] + b_ref[...]
        o_ref[...] = jnp.concatenate(
            [_leaky(out) + hb, jnp.zeros((N_NODES, LANES - HID), jnp.float32)],
            axis=1)

    return pl.pallas_call(
        body, out_shape=jax.ShapeDtypeStruct((N_NODES, LANES), jnp.float32),
    )(acc2, h128, root, root_b, ln_g, ln_b)


def _tc_pool_head(h128, batch2, W1, b1, W2, b2, W3, b3):
    def body(h_ref, bt_ref, w1_ref, b1_ref, w2_ref, b2_ref, w3_ref, b3_ref, o_ref):
        gids = lax.broadcasted_iota(jnp.int32, (N_GRAPHS, N_NODES), 0)
        mask = (bt_ref[...] == gids).astype(jnp.float32)
        cnt = jnp.maximum(jnp.sum(mask, axis=1, keepdims=True), 1.0)
        pooled = jnp.dot(mask, h_ref[:, :HID],
                         preferred_element_type=jnp.float32) / cnt
        z = _leaky(jnp.dot(pooled, w1_ref[...],
                           preferred_element_type=jnp.float32) + b1_ref[...])
        z = _leaky(jnp.dot(z, w2_ref[...],
                           preferred_element_type=jnp.float32) + b2_ref[...])
        o_ref[...] = jnp.dot(z, w3_ref[...],
                             preferred_element_type=jnp.float32) + b3_ref[...]

    return pl.pallas_call(
        body, out_shape=jax.ShapeDtypeStruct((N_GRAPHS, 1), jnp.float32),
    )(h128, batch2, W1, b1, W2, b2, W3, b3)


def kernel(x, edge_index, edge_attr, batch, params):
    src = edge_index[0]
    dst = edge_index[1]
    src_p = jnp.pad(src, (0, EP - N_EDGES))
    dst_p = jnp.pad(dst, (0, EP - N_EDGES))
    ea_p = jnp.pad(edge_attr, ((0, EP - N_EDGES), (0, 0)))
    batch2 = batch.reshape(1, N_NODES)
    zero_rows = jnp.zeros((ROWS_PER_TILE, MSGW), jnp.float32)

    def r2(v):
        return v.reshape(1, -1)

    h = _tc_prep(x, params["Wp"], r2(params["bp"]))
    for blk in params["blocks"]:
        e2 = _tc_edge_mlp(ea_p, blk["W1"], r2(blk["b1"]), blk["W2"], r2(blk["b2"]))
        u = _sc_gather(h, src_p)
        msgp = _tc_message(e2, u, blk["W3"].reshape(HID * HID, HID),
                           blk["b3"].reshape(HID, HID))
        acc2 = _sc_scatter(msgp, dst_p, zero_rows)
        h = _tc_update(acc2[:, :N_NODES, :], h, blk["root"], r2(blk["root_b"]),
                       r2(blk["ln_g"]), r2(blk["ln_b"]))
    hd = params["head"]
    pred = _tc_pool_head(h, batch2, hd["W1"], r2(hd["b1"]), hd["W2"],
                         r2(hd["b2"]), hd["W3"], r2(hd["b3"]))
    return pred.reshape(N_GRAPHS)


# speedup vs baseline: 1.8116x; 1.8116x over previous
import functools

import jax
import jax.numpy as jnp
from jax import lax
from jax.experimental import pallas as pl
from jax.experimental.pallas import tpu as pltpu, tpu_sc as plsc

N_NODES = 10000
N_EDGES = 160000
N_GRAPHS = 16
D_IN = 4
D_EDGE = 5
HID = 32

NC = 2
NS = 16
NW = NC * NS
CHUNK = 128
CHUNKS_PER_W = 40
EP = NW * CHUNKS_PER_W * CHUNK
E_PER_W = CHUNKS_PER_W * CHUNK
MSGW = 128
NP_ROWS = 10240
ROWS_PER_TILE = NP_ROWS // NS


def _leaky(v):
    return jnp.where(v >= 0, v, 0.1 * v)


_MESH = plsc.VectorSubcoreMesh(core_axis_name="c", subcore_axis_name="s")


LANES = 128


@functools.partial(
    pl.kernel,
    out_type=jax.ShapeDtypeStruct((EP, LANES), jnp.float32),
    mesh=_MESH,
    scratch_types=[
        pltpu.VMEM((CHUNK,), jnp.int32),
        pltpu.VMEM((CHUNK, LANES), jnp.float32),
        pltpu.SemaphoreType.DMA,
    ],
)
def _sc_gather(h_hbm, src_hbm, u_hbm, idx_v, rows_v, sem):
    wid = lax.axis_index("s") * NC + lax.axis_index("c")
    base = wid * E_PER_W

    def body(c, _):
        off = base + c * CHUNK
        pltpu.sync_copy(src_hbm.at[pl.ds(off, CHUNK)], idx_v)
        pltpu.async_copy(h_hbm.at[idx_v], rows_v, sem).wait()
        pltpu.sync_copy(rows_v, u_hbm.at[pl.ds(off, CHUNK)])
        return _

    lax.fori_loop(0, CHUNKS_PER_W, body, None)


@functools.partial(
    pl.kernel,
    out_type=jax.ShapeDtypeStruct((NC, NP_ROWS, MSGW), jnp.float32),
    mesh=_MESH,
    scratch_types=[
        pltpu.VMEM((CHUNK,), jnp.int32),
        pltpu.VMEM((CHUNK, MSGW), jnp.float32),
        pltpu.VMEM_SHARED((NP_ROWS, MSGW), jnp.float32),
    ],
)
def _sc_scatter(msg_hbm, dst_hbm, zero_hbm, acc_hbm, idx_v, msg_v, shared):
    cid = lax.axis_index("c")
    sid = lax.axis_index("s")
    wid = sid * NC + cid
    base = wid * E_PER_W
    row0 = sid * ROWS_PER_TILE

    pltpu.sync_copy(zero_hbm, shared.at[pl.ds(row0, ROWS_PER_TILE)])
    plsc.subcore_barrier()

    def body(c, _):
        off = base + c * CHUNK
        pltpu.sync_copy(dst_hbm.at[pl.ds(off, CHUNK)], idx_v)
        pltpu.sync_copy(msg_hbm.at[pl.ds(off, CHUNK)], msg_v)
        pltpu.sync_copy(msg_v, shared.at[idx_v], add=True)
        return _

    lax.fori_loop(0, CHUNKS_PER_W, body, None)
    plsc.subcore_barrier()
    pltpu.sync_copy(shared.at[pl.ds(row0, ROWS_PER_TILE)],
                    acc_hbm.at[cid, pl.ds(row0, ROWS_PER_TILE)])


def _tc_prep(x, Wp, bp):
    def body(x_ref, w_ref, b_ref, o_ref):
        h = _leaky(
            jnp.dot(x_ref[...], w_ref[...], preferred_element_type=jnp.float32)
            + b_ref[...])
        o_ref[...] = jnp.concatenate(
            [h, jnp.zeros((N_NODES, LANES - HID), jnp.float32)], axis=1)
    return pl.pallas_call(
        body, out_shape=jax.ShapeDtypeStruct((N_NODES, LANES), jnp.float32),
    )(x, Wp, bp)


def _tc_edge_mlp(ea_p, W1, b1, W2, b2):
    BLK = 2048
    grid = EP // BLK

    def body(ea_ref, w1_ref, b1_ref, w2_ref, b2_ref, o_ref):
        e1 = _leaky(jnp.dot(ea_ref[...], w1_ref[...],
                            preferred_element_type=jnp.float32) + b1_ref[...])
        o_ref[...] = _leaky(jnp.dot(e1, w2_ref[...],
                                    preferred_element_type=jnp.float32) + b2_ref[...])

    return pl.pallas_call(
        body,
        grid=(grid,),
        in_specs=[
            pl.BlockSpec((BLK, D_EDGE), lambda i: (i, 0)),
            pl.BlockSpec((D_EDGE, HID), lambda i: (0, 0)),
            pl.BlockSpec((1, HID), lambda i: (0, 0)),
            pl.BlockSpec((HID, HID), lambda i: (0, 0)),
            pl.BlockSpec((1, HID), lambda i: (0, 0)),
        ],
        out_specs=pl.BlockSpec((BLK, HID), lambda i: (i, 0)),
        out_shape=jax.ShapeDtypeStruct((EP, HID), jnp.float32),
    )(ea_p, W1, b1, W2, b2)


def _tc_message(e2, u, W3r, B3, Te, Tu):
    BLK = 512
    grid = EP // BLK

    def body(e2_ref, u_ref, w3_ref, b3_ref, te_ref, tu_ref, o_ref):
        e2b = e2_ref[...]
        ub = u_ref[:, :HID]
        ze = jnp.dot(e2b, te_ref[...], preferred_element_type=jnp.float32)
        zu = jnp.dot(ub, tu_ref[...], preferred_element_type=jnp.float32)
        z = ze * zu
        msg = (jnp.dot(z, w3_ref[...], preferred_element_type=jnp.float32)
               + jnp.dot(ub, b3_ref[...], preferred_element_type=jnp.float32))
        rows = (pl.program_id(0) * BLK
                + lax.broadcasted_iota(jnp.int32, (BLK, 1), 0))
        maskf = (rows < N_EDGES).astype(jnp.float32)
        o_ref[...] = jnp.concatenate(
            [msg * maskf, maskf, jnp.zeros((BLK, MSGW - HID - 1), jnp.float32)],
            axis=1)

    return pl.pallas_call(
        body,
        grid=(grid,),
        in_specs=[
            pl.BlockSpec((BLK, HID), lambda i: (i, 0)),
            pl.BlockSpec((BLK, LANES), lambda i: (i, 0)),
            pl.BlockSpec((HID * HID, HID), lambda i: (0, 0)),
            pl.BlockSpec((HID, HID), lambda i: (0, 0)),
            pl.BlockSpec((HID, HID * HID), lambda i: (0, 0)),
            pl.BlockSpec((HID, HID * HID), lambda i: (0, 0)),
        ],
        out_specs=pl.BlockSpec((BLK, MSGW), lambda i: (i, 0)),
        out_shape=jax.ShapeDtypeStruct((EP, MSGW), jnp.float32),
    )(e2, u, W3r, B3, Te, Tu)


def _tc_update(acc2, h128, root, root_b, ln_g, ln_b):
    def body(a_ref, h_ref, r_ref, rb_ref, g_ref, b_ref, o_ref):
        s = a_ref[0] + a_ref[1]
        deg = jnp.maximum(s[:, HID:HID + 1], 1.0)
        hb = h_ref[:, :HID]
        out = (s[:, :HID] / deg
               + jnp.dot(hb, r_ref[...], preferred_element_type=jnp.float32)
               + rb_ref[...])
        mu = jnp.mean(out, axis=-1, keepdims=True)
        var = jnp.mean((out - mu) ** 2, axis=-1, keepdims=True)
        out = (out - mu) / jnp.sqrt(var + 1e-5) * g_ref[...] + b_ref[...]
        o_ref[...] = jnp.concatenate(
            [_leaky(out) + hb, jnp.zeros((N_NODES, LANES - HID), jnp.float32)],
            axis=1)

    return pl.pallas_call(
        body, out_shape=jax.ShapeDtypeStruct((N_NODES, LANES), jnp.float32),
    )(acc2, h128, root, root_b, ln_g, ln_b)


def _tc_pool_head(h128, batch2, W1, b1, W2, b2, W3, b3):
    def body(h_ref, bt_ref, w1_ref, b1_ref, w2_ref, b2_ref, w3_ref, b3_ref, o_ref):
        gids = lax.broadcasted_iota(jnp.int32, (N_GRAPHS, N_NODES), 0)
        mask = (bt_ref[...] == gids).astype(jnp.float32)
        cnt = jnp.maximum(jnp.sum(mask, axis=1, keepdims=True), 1.0)
        pooled = jnp.dot(mask, h_ref[:, :HID],
                         preferred_element_type=jnp.float32) / cnt
        z = _leaky(jnp.dot(pooled, w1_ref[...],
                           preferred_element_type=jnp.float32) + b1_ref[...])
        z = _leaky(jnp.dot(z, w2_ref[...],
                           preferred_element_type=jnp.float32) + b2_ref[...])
        o_ref[...] = jnp.dot(z, w3_ref[...],
                             preferred_element_type=jnp.float32) + b3_ref[...]

    return pl.pallas_call(
        body, out_shape=jax.ShapeDtypeStruct((N_GRAPHS, 1), jnp.float32),
    )(h128, batch2, W1, b1, W2, b2, W3, b3)


def kernel(x, edge_index, edge_attr, batch, params):
    src = edge_index[0]
    dst = edge_index[1]
    src_p = jnp.pad(src, (0, EP - N_EDGES))
    dst_p = jnp.pad(dst, (0, EP - N_EDGES))
    ea_p = jnp.pad(edge_attr, ((0, EP - N_EDGES), (0, 0)))
    batch2 = batch.reshape(1, N_NODES)
    zero_rows = jnp.zeros((ROWS_PER_TILE, MSGW), jnp.float32)
    eye = jnp.eye(HID, dtype=jnp.float32)
    Te = jnp.repeat(eye, HID, axis=1)
    Tu = jnp.tile(eye, (1, HID))

    def r2(v):
        return v.reshape(1, -1)

    h = _tc_prep(x, params["Wp"], r2(params["bp"]))
    for blk in params["blocks"]:
        e2 = _tc_edge_mlp(ea_p, blk["W1"], r2(blk["b1"]), blk["W2"], r2(blk["b2"]))
        u = _sc_gather(h, src_p)
        msgp = _tc_message(e2, u, blk["W3"].reshape(HID * HID, HID),
                           blk["b3"].reshape(HID, HID), Te, Tu)
        acc2 = _sc_scatter(msgp, dst_p, zero_rows)
        h = _tc_update(acc2[:, :N_NODES, :], h, blk["root"], r2(blk["root_b"]),
                       r2(blk["ln_g"]), r2(blk["ln_b"]))
    hd = params["head"]
    pred = _tc_pool_head(h, batch2, hd["W1"], r2(hd["b1"]), hd["W2"],
                         r2(hd["b2"]), hd["W3"], r2(hd["b3"]))
    return pred.reshape(N_GRAPHS)


# speedup vs baseline: 1.9643x; 1.0843x over previous
import functools

import jax
import jax.numpy as jnp
from jax import lax
from jax.experimental import pallas as pl
from jax.experimental.pallas import tpu as pltpu, tpu_sc as plsc

N_NODES = 10000
N_EDGES = 160000
N_GRAPHS = 16
D_IN = 4
D_EDGE = 5
HID = 32

NC = 2
NS = 16
NW = NC * NS
CHUNK = 128
CHUNKS_PER_W = 40
EP = NW * CHUNKS_PER_W * CHUNK
E_PER_W = CHUNKS_PER_W * CHUNK
MSGW = 128
NP_ROWS = 10240
ROWS_PER_TILE = NP_ROWS // NS


def _leaky(v):
    return jnp.where(v >= 0, v, 0.1 * v)


_MESH = plsc.VectorSubcoreMesh(core_axis_name="c", subcore_axis_name="s")


LANES = 128


NBUF = 4
GROUPS = CHUNKS_PER_W // NBUF


@functools.partial(
    pl.kernel,
    out_type=jax.ShapeDtypeStruct((EP, LANES), jnp.float32),
    mesh=_MESH,
    scratch_types=[
        pltpu.VMEM((E_PER_W,), jnp.int32),
        pltpu.VMEM((NBUF, CHUNK, LANES), jnp.float32),
    ] + [pltpu.SemaphoreType.DMA] * (2 * NBUF),
)
def _sc_gather(h_hbm, src_hbm, u_hbm, idx_v, rows_v, *sems):
    gsems, ssems = sems[:NBUF], sems[NBUF:]
    wid = lax.axis_index("s") * NC + lax.axis_index("c")
    base = wid * E_PER_W
    pltpu.sync_copy(src_hbm.at[pl.ds(base, E_PER_W)], idx_v)

    def group(g, _):
        gh = [pltpu.async_copy(
                  h_hbm.at[idx_v.at[pl.ds((g * NBUF + b) * CHUNK, CHUNK)]],
                  rows_v.at[b], gsems[b])
              for b in range(NBUF)]
        sh = []
        for b in range(NBUF):
            gh[b].wait()
            off = base + (g * NBUF + b) * CHUNK
            sh.append(pltpu.async_copy(rows_v.at[b],
                                       u_hbm.at[pl.ds(off, CHUNK)], ssems[b]))
        for b in range(NBUF):
            sh[b].wait()
        return _

    lax.fori_loop(0, GROUPS, group, None)


@functools.partial(
    pl.kernel,
    out_type=jax.ShapeDtypeStruct((NC, NP_ROWS, MSGW), jnp.float32),
    mesh=_MESH,
    scratch_types=[
        pltpu.VMEM((CHUNKS_PER_W, CHUNK), jnp.int32),
        pltpu.VMEM((2, CHUNK, MSGW), jnp.float32),
        pltpu.VMEM_SHARED((NP_ROWS, MSGW), jnp.float32),
    ] + [pltpu.SemaphoreType.DMA] * 4,
)
def _sc_scatter(msg_hbm, dst3_hbm, zero_hbm, acc_hbm, idx_v, msg_v, shared, *sems):
    nb = 2
    lsems, scsems = sems[:nb], sems[nb:]
    cid = lax.axis_index("c")
    sid = lax.axis_index("s")
    wid = sid * NC + cid
    base = wid * E_PER_W
    row0 = sid * ROWS_PER_TILE

    pltpu.sync_copy(zero_hbm, shared.at[pl.ds(row0, ROWS_PER_TILE)])
    pltpu.sync_copy(dst3_hbm.at[wid], idx_v)
    plsc.subcore_barrier()

    def group(g, _):
        lh = [pltpu.async_copy(
                  msg_hbm.at[pl.ds(base + (g * nb + b) * CHUNK, CHUNK)],
                  msg_v.at[b], lsems[b])
              for b in range(nb)]
        sh = []
        for b in range(nb):
            lh[b].wait()
            sh.append(pltpu.async_copy(msg_v.at[b],
                                       shared.at[idx_v.at[g * nb + b]],
                                       scsems[b], add=True))
        for b in range(nb):
            sh[b].wait()
        return _

    lax.fori_loop(0, CHUNKS_PER_W // nb, group, None)
    plsc.subcore_barrier()
    pltpu.sync_copy(shared.at[pl.ds(row0, ROWS_PER_TILE)],
                    acc_hbm.at[cid, pl.ds(row0, ROWS_PER_TILE)])


def _tc_prep(x, Wp, bp):
    def body(x_ref, w_ref, b_ref, o_ref):
        h = _leaky(
            jnp.dot(x_ref[...], w_ref[...], preferred_element_type=jnp.float32)
            + b_ref[...])
        o_ref[...] = jnp.concatenate(
            [h, jnp.zeros((N_NODES, LANES - HID), jnp.float32)], axis=1)
    return pl.pallas_call(
        body, out_shape=jax.ShapeDtypeStruct((N_NODES, LANES), jnp.float32),
    )(x, Wp, bp)


def _tc_edge_mlp(ea_p, W1, b1, W2, b2):
    BLK = 2048
    grid = EP // BLK

    def body(ea_ref, w1_ref, b1_ref, w2_ref, b2_ref, o_ref):
        e1 = _leaky(jnp.dot(ea_ref[...], w1_ref[...],
                            preferred_element_type=jnp.float32) + b1_ref[...])
        o_ref[...] = _leaky(jnp.dot(e1, w2_ref[...],
                                    preferred_element_type=jnp.float32) + b2_ref[...])

    return pl.pallas_call(
        body,
        grid=(grid,),
        in_specs=[
            pl.BlockSpec((BLK, D_EDGE), lambda i: (i, 0)),
            pl.BlockSpec((D_EDGE, HID), lambda i: (0, 0)),
            pl.BlockSpec((1, HID), lambda i: (0, 0)),
            pl.BlockSpec((HID, HID), lambda i: (0, 0)),
            pl.BlockSpec((1, HID), lambda i: (0, 0)),
        ],
        out_specs=pl.BlockSpec((BLK, HID), lambda i: (i, 0)),
        out_shape=jax.ShapeDtypeStruct((EP, HID), jnp.float32),
    )(ea_p, W1, b1, W2, b2)


def _tc_message(e2, u, W3r, B3, Te, Tu):
    BLK = 512
    grid = EP // BLK

    def body(e2_ref, u_ref, w3_ref, b3_ref, te_ref, tu_ref, o_ref):
        e2b = e2_ref[...]
        ub = u_ref[:, :HID]
        ze = jnp.dot(e2b, te_ref[...], preferred_element_type=jnp.float32)
        zu = jnp.dot(ub, tu_ref[...], preferred_element_type=jnp.float32)
        z = ze * zu
        msg = (jnp.dot(z, w3_ref[...], preferred_element_type=jnp.float32)
               + jnp.dot(ub, b3_ref[...], preferred_element_type=jnp.float32))
        rows = (pl.program_id(0) * BLK
                + lax.broadcasted_iota(jnp.int32, (BLK, 1), 0))
        maskf = (rows < N_EDGES).astype(jnp.float32)
        o_ref[...] = jnp.concatenate(
            [msg * maskf, maskf, jnp.zeros((BLK, MSGW - HID - 1), jnp.float32)],
            axis=1)

    return pl.pallas_call(
        body,
        grid=(grid,),
        in_specs=[
            pl.BlockSpec((BLK, HID), lambda i: (i, 0)),
            pl.BlockSpec((BLK, LANES), lambda i: (i, 0)),
            pl.BlockSpec((HID * HID, HID), lambda i: (0, 0)),
            pl.BlockSpec((HID, HID), lambda i: (0, 0)),
            pl.BlockSpec((HID, HID * HID), lambda i: (0, 0)),
            pl.BlockSpec((HID, HID * HID), lambda i: (0, 0)),
        ],
        out_specs=pl.BlockSpec((BLK, MSGW), lambda i: (i, 0)),
        out_shape=jax.ShapeDtypeStruct((EP, MSGW), jnp.float32),
    )(e2, u, W3r, B3, Te, Tu)


def _tc_update(acc2, h128, root, root_b, ln_g, ln_b):
    def body(a_ref, h_ref, r_ref, rb_ref, g_ref, b_ref, o_ref):
        s = a_ref[0] + a_ref[1]
        deg = jnp.maximum(s[:, HID:HID + 1], 1.0)
        hb = h_ref[:, :HID]
        out = (s[:, :HID] / deg
               + jnp.dot(hb, r_ref[...], preferred_element_type=jnp.float32)
               + rb_ref[...])
        mu = jnp.mean(out, axis=-1, keepdims=True)
        var = jnp.mean((out - mu) ** 2, axis=-1, keepdims=True)
        out = (out - mu) / jnp.sqrt(var + 1e-5) * g_ref[...] + b_ref[...]
        o_ref[...] = jnp.concatenate(
            [_leaky(out) + hb, jnp.zeros((N_NODES, LANES - HID), jnp.float32)],
            axis=1)

    return pl.pallas_call(
        body, out_shape=jax.ShapeDtypeStruct((N_NODES, LANES), jnp.float32),
    )(acc2, h128, root, root_b, ln_g, ln_b)


def _tc_pool_head(h128, batch2, W1, b1, W2, b2, W3, b3):
    def body(h_ref, bt_ref, w1_ref, b1_ref, w2_ref, b2_ref, w3_ref, b3_ref, o_ref):
        gids = lax.broadcasted_iota(jnp.int32, (N_GRAPHS, N_NODES), 0)
        mask = (bt_ref[...] == gids).astype(jnp.float32)
        cnt = jnp.maximum(jnp.sum(mask, axis=1, keepdims=True), 1.0)
        pooled = jnp.dot(mask, h_ref[:, :HID],
                         preferred_element_type=jnp.float32) / cnt
        z = _leaky(jnp.dot(pooled, w1_ref[...],
                           preferred_element_type=jnp.float32) + b1_ref[...])
        z = _leaky(jnp.dot(z, w2_ref[...],
                           preferred_element_type=jnp.float32) + b2_ref[...])
        o_ref[...] = jnp.dot(z, w3_ref[...],
                             preferred_element_type=jnp.float32) + b3_ref[...]

    return pl.pallas_call(
        body, out_shape=jax.ShapeDtypeStruct((N_GRAPHS, 1), jnp.float32),
    )(h128, batch2, W1, b1, W2, b2, W3, b3)


def kernel(x, edge_index, edge_attr, batch, params):
    src = edge_index[0]
    dst = edge_index[1]
    src_p = jnp.pad(src, (0, EP - N_EDGES))
    dst3 = jnp.pad(dst, (0, EP - N_EDGES)).reshape(NW, CHUNKS_PER_W, CHUNK)
    ea_p = jnp.pad(edge_attr, ((0, EP - N_EDGES), (0, 0)))
    batch2 = batch.reshape(1, N_NODES)
    zero_rows = jnp.zeros((ROWS_PER_TILE, MSGW), jnp.float32)
    eye = jnp.eye(HID, dtype=jnp.float32)
    Te = jnp.repeat(eye, HID, axis=1)
    Tu = jnp.tile(eye, (1, HID))

    def r2(v):
        return v.reshape(1, -1)

    h = _tc_prep(x, params["Wp"], r2(params["bp"]))
    for blk in params["blocks"]:
        e2 = _tc_edge_mlp(ea_p, blk["W1"], r2(blk["b1"]), blk["W2"], r2(blk["b2"]))
        u = _sc_gather(h, src_p)
        msgp = _tc_message(e2, u, blk["W3"].reshape(HID * HID, HID),
                           blk["b3"].reshape(HID, HID), Te, Tu)
        acc2 = _sc_scatter(msgp, dst3, zero_rows)
        h = _tc_update(acc2[:, :N_NODES, :], h, blk["root"], r2(blk["root_b"]),
                       r2(blk["ln_g"]), r2(blk["ln_b"]))
    hd = params["head"]
    pred = _tc_pool_head(h, batch2, hd["W1"], r2(hd["b1"]), hd["W2"],
                         r2(hd["b2"]), hd["W3"], r2(hd["b3"]))
    return pred.reshape(N_GRAPHS)


# speedup vs baseline: 2.1726x; 1.1060x over previous
import functools

import jax
import jax.numpy as jnp
from jax import lax
from jax.experimental import pallas as pl
from jax.experimental.pallas import tpu as pltpu, tpu_sc as plsc

N_NODES = 10000
N_EDGES = 160000
N_GRAPHS = 16
D_IN = 4
D_EDGE = 5
HID = 32

NC = 2
NS = 16
NW = NC * NS
CHUNK = 128
CHUNKS_PER_W = 40
EP = NW * CHUNKS_PER_W * CHUNK
E_PER_W = CHUNKS_PER_W * CHUNK
MSGW = 128
NP_ROWS = 10240
ROWS_PER_TILE = NP_ROWS // NS


def _leaky(v):
    return jnp.where(v >= 0, v, 0.1 * v)


_MESH = plsc.VectorSubcoreMesh(core_axis_name="c", subcore_axis_name="s")


LANES = 128


NBUF = 4
GROUPS = CHUNKS_PER_W // NBUF


@functools.partial(
    pl.kernel,
    out_type=jax.ShapeDtypeStruct((EP, LANES), jnp.float32),
    mesh=_MESH,
    scratch_types=[
        pltpu.VMEM((E_PER_W,), jnp.int32),
        pltpu.VMEM((NBUF, CHUNK, LANES), jnp.float32),
    ] + [pltpu.SemaphoreType.DMA] * (2 * NBUF),
)
def _sc_gather(h_hbm, src_hbm, u_hbm, idx_v, rows_v, *sems):
    gsems, ssems = sems[:NBUF], sems[NBUF:]
    wid = lax.axis_index("s") * NC + lax.axis_index("c")
    base = wid * E_PER_W
    pltpu.sync_copy(src_hbm.at[pl.ds(base, E_PER_W)], idx_v)

    def group(g, _):
        gh = [pltpu.async_copy(
                  h_hbm.at[idx_v.at[pl.ds((g * NBUF + b) * CHUNK, CHUNK)]],
                  rows_v.at[b], gsems[b])
              for b in range(NBUF)]
        sh = []
        for b in range(NBUF):
            gh[b].wait()
            off = base + (g * NBUF + b) * CHUNK
            sh.append(pltpu.async_copy(rows_v.at[b],
                                       u_hbm.at[pl.ds(off, CHUNK)], ssems[b]))
        for b in range(NBUF):
            sh[b].wait()
        return _

    lax.fori_loop(0, GROUPS, group, None)


@functools.partial(
    pl.kernel,
    out_type=jax.ShapeDtypeStruct((NC, NP_ROWS, MSGW), jnp.float32),
    mesh=_MESH,
    scratch_types=[
        pltpu.VMEM((CHUNKS_PER_W, CHUNK), jnp.int32),
        pltpu.VMEM((2, CHUNK, MSGW), jnp.float32),
        pltpu.VMEM_SHARED((NP_ROWS, MSGW), jnp.float32),
    ] + [pltpu.SemaphoreType.DMA] * 4,
)
def _sc_scatter(msg_hbm, dst3_hbm, zero_hbm, acc_hbm, idx_v, msg_v, shared, *sems):
    nb = 2
    lsems, scsems = sems[:nb], sems[nb:]
    cid = lax.axis_index("c")
    sid = lax.axis_index("s")
    wid = sid * NC + cid
    base = wid * E_PER_W
    row0 = sid * ROWS_PER_TILE

    pltpu.sync_copy(zero_hbm, shared.at[pl.ds(row0, ROWS_PER_TILE)])
    pltpu.sync_copy(dst3_hbm.at[wid], idx_v)
    plsc.subcore_barrier()

    def group(g, _):
        lh = [pltpu.async_copy(
                  msg_hbm.at[pl.ds(base + (g * nb + b) * CHUNK, CHUNK)],
                  msg_v.at[b], lsems[b])
              for b in range(nb)]
        sh = []
        for b in range(nb):
            lh[b].wait()
            sh.append(pltpu.async_copy(msg_v.at[b],
                                       shared.at[idx_v.at[g * nb + b]],
                                       scsems[b], add=True))
        for b in range(nb):
            sh[b].wait()
        return _

    lax.fori_loop(0, CHUNKS_PER_W // nb, group, None)
    plsc.subcore_barrier()
    pltpu.sync_copy(shared.at[pl.ds(row0, ROWS_PER_TILE)],
                    acc_hbm.at[cid, pl.ds(row0, ROWS_PER_TILE)])


def _tc_prep(x, Wp, bp):
    def body(x_ref, w_ref, b_ref, o_ref):
        h = _leaky(
            jnp.dot(x_ref[...], w_ref[...], preferred_element_type=jnp.float32)
            + b_ref[...])
        o_ref[...] = jnp.concatenate(
            [h, jnp.zeros((N_NODES, LANES - HID), jnp.float32)], axis=1)
    return pl.pallas_call(
        body, out_shape=jax.ShapeDtypeStruct((N_NODES, LANES), jnp.float32),
    )(x, Wp, bp)


def _tc_message(ea_p, u, W1, b1, W2, b2, W3r, B3, Te, Tu):
    BLK = 1024
    grid = EP // BLK

    def body(ea_ref, u_ref, w1_ref, b1_ref, w2_ref, b2_ref,
             w3_ref, b3_ref, te_ref, tu_ref, o_ref):
        e1 = _leaky(jnp.dot(ea_ref[...], w1_ref[...],
                            preferred_element_type=jnp.float32) + b1_ref[...])
        e2 = _leaky(jnp.dot(e1, w2_ref[...],
                            preferred_element_type=jnp.float32) + b2_ref[...])
        ub = u_ref[:, :HID]
        ze = jnp.dot(e2.astype(jnp.bfloat16), te_ref[...],
                     preferred_element_type=jnp.float32)
        zu = jnp.dot(ub.astype(jnp.bfloat16), tu_ref[...],
                     preferred_element_type=jnp.float32)
        z = (ze * zu).astype(jnp.bfloat16)
        msg = (jnp.dot(z, w3_ref[...], preferred_element_type=jnp.float32)
               + jnp.dot(ub, b3_ref[...], preferred_element_type=jnp.float32))
        rows = (pl.program_id(0) * BLK
                + lax.broadcasted_iota(jnp.int32, (BLK, 1), 0))
        maskf = (rows < N_EDGES).astype(jnp.float32)
        o_ref[...] = jnp.concatenate(
            [msg * maskf, maskf, jnp.zeros((BLK, MSGW - HID - 1), jnp.float32)],
            axis=1)

    return pl.pallas_call(
        body,
        grid=(grid,),
        in_specs=[
            pl.BlockSpec((BLK, D_EDGE), lambda i: (i, 0)),
            pl.BlockSpec((BLK, LANES), lambda i: (i, 0)),
            pl.BlockSpec((D_EDGE, HID), lambda i: (0, 0)),
            pl.BlockSpec((1, HID), lambda i: (0, 0)),
            pl.BlockSpec((HID, HID), lambda i: (0, 0)),
            pl.BlockSpec((1, HID), lambda i: (0, 0)),
            pl.BlockSpec((HID * HID, HID), lambda i: (0, 0)),
            pl.BlockSpec((HID, HID), lambda i: (0, 0)),
            pl.BlockSpec((HID, HID * HID), lambda i: (0, 0)),
            pl.BlockSpec((HID, HID * HID), lambda i: (0, 0)),
        ],
        out_specs=pl.BlockSpec((BLK, MSGW), lambda i: (i, 0)),
        out_shape=jax.ShapeDtypeStruct((EP, MSGW), jnp.float32),
    )(ea_p, u, W1, b1, W2, b2, W3r, B3, Te, Tu)


def _tc_update(acc2, h128, root, root_b, ln_g, ln_b):
    def body(a_ref, h_ref, r_ref, rb_ref, g_ref, b_ref, o_ref):
        s = a_ref[0] + a_ref[1]
        deg = jnp.maximum(s[:, HID:HID + 1], 1.0)
        hb = h_ref[:, :HID]
        out = (s[:, :HID] / deg
               + jnp.dot(hb, r_ref[...], preferred_element_type=jnp.float32)
               + rb_ref[...])
        mu = jnp.mean(out, axis=-1, keepdims=True)
        var = jnp.mean((out - mu) ** 2, axis=-1, keepdims=True)
        out = (out - mu) / jnp.sqrt(var + 1e-5) * g_ref[...] + b_ref[...]
        o_ref[...] = jnp.concatenate(
            [_leaky(out) + hb, jnp.zeros((N_NODES, LANES - HID), jnp.float32)],
            axis=1)

    return pl.pallas_call(
        body, out_shape=jax.ShapeDtypeStruct((N_NODES, LANES), jnp.float32),
    )(acc2, h128, root, root_b, ln_g, ln_b)


def _tc_update_pool_head(acc2, h128, root, root_b, ln_g, ln_b,
                         batch2, W1, b1, W2, b2, W3, b3):
    def body(a_ref, h_ref, r_ref, rb_ref, g_ref, b_ref, bt_ref,
             w1_ref, b1_ref, w2_ref, b2_ref, w3_ref, b3_ref, o_ref):
        s = a_ref[0] + a_ref[1]
        deg = jnp.maximum(s[:, HID:HID + 1], 1.0)
        hb = h_ref[:, :HID]
        out = (s[:, :HID] / deg
               + jnp.dot(hb, r_ref[...], preferred_element_type=jnp.float32)
               + rb_ref[...])
        mu = jnp.mean(out, axis=-1, keepdims=True)
        var = jnp.mean((out - mu) ** 2, axis=-1, keepdims=True)
        out = (out - mu) / jnp.sqrt(var + 1e-5) * g_ref[...] + b_ref[...]
        hn = _leaky(out) + hb
        gids = lax.broadcasted_iota(jnp.int32, (N_GRAPHS, N_NODES), 0)
        mask = (bt_ref[...] == gids).astype(jnp.float32)
        cnt = jnp.maximum(jnp.sum(mask, axis=1, keepdims=True), 1.0)
        pooled = jnp.dot(mask, hn, preferred_element_type=jnp.float32) / cnt
        z = _leaky(jnp.dot(pooled, w1_ref[...],
                           preferred_element_type=jnp.float32) + b1_ref[...])
        z = _leaky(jnp.dot(z, w2_ref[...],
                           preferred_element_type=jnp.float32) + b2_ref[...])
        o_ref[...] = jnp.dot(z, w3_ref[...],
                             preferred_element_type=jnp.float32) + b3_ref[...]

    return pl.pallas_call(
        body, out_shape=jax.ShapeDtypeStruct((N_GRAPHS, 1), jnp.float32),
    )(acc2, h128, root, root_b, ln_g, ln_b, batch2, W1, b1, W2, b2, W3, b3)


def kernel(x, edge_index, edge_attr, batch, params):
    src = edge_index[0]
    dst = edge_index[1]
    src_p = jnp.pad(src, (0, EP - N_EDGES))
    dst3 = jnp.pad(dst, (0, EP - N_EDGES)).reshape(NW, CHUNKS_PER_W, CHUNK)
    ea_p = jnp.pad(edge_attr, ((0, EP - N_EDGES), (0, 0)))
    batch2 = batch.reshape(1, N_NODES)
    zero_rows = jnp.zeros((ROWS_PER_TILE, MSGW), jnp.float32)
    eye = jnp.eye(HID, dtype=jnp.bfloat16)
    Te = jnp.repeat(eye, HID, axis=1)
    Tu = jnp.tile(eye, (1, HID))

    def r2(v):
        return v.reshape(1, -1)

    def msg_args(blk):
        return (blk["W1"], r2(blk["b1"]), blk["W2"], r2(blk["b2"]),
                blk["W3"].reshape(HID * HID, HID).astype(jnp.bfloat16),
                blk["b3"].reshape(HID, HID), Te, Tu)

    hd = params["head"]
    blk0, blk1 = params["blocks"]
    h = _tc_prep(x, params["Wp"], r2(params["bp"]))
    u = _sc_gather(h, src_p)
    msgp = _tc_message(ea_p, u, *msg_args(blk0))
    acc2 = _sc_scatter(msgp, dst3, zero_rows)
    h = _tc_update(acc2[:, :N_NODES, :], h, blk0["root"], r2(blk0["root_b"]),
                   r2(blk0["ln_g"]), r2(blk0["ln_b"]))
    u = _sc_gather(h, src_p)
    msgp = _tc_message(ea_p, u, *msg_args(blk1))
    acc2 = _sc_scatter(msgp, dst3, zero_rows)
    pred = _tc_update_pool_head(
        acc2[:, :N_NODES, :], h, blk1["root"], r2(blk1["root_b"]),
        r2(blk1["ln_g"]), r2(blk1["ln_b"]), batch2,
        hd["W1"], r2(hd["b1"]), hd["W2"], r2(hd["b2"]), hd["W3"], r2(hd["b3"]))
    return pred.reshape(N_GRAPHS)


# speedup vs baseline: 2.1806x; 1.0037x over previous
import functools

import jax
import jax.numpy as jnp
from jax import lax
from jax.experimental import pallas as pl
from jax.experimental.pallas import tpu as pltpu, tpu_sc as plsc

N_NODES = 10000
N_EDGES = 160000
N_GRAPHS = 16
D_IN = 4
D_EDGE = 5
HID = 32

NC = 2
NS = 16
NW = NC * NS
CHUNK = 128
CHUNKS_PER_W = 40
EP = NW * CHUNKS_PER_W * CHUNK
E_PER_W = CHUNKS_PER_W * CHUNK
MSGW = 128
NP_ROWS = 10240
ROWS_PER_TILE = NP_ROWS // NS


def _leaky(v):
    return jnp.where(v >= 0, v, 0.1 * v)


_MESH = plsc.VectorSubcoreMesh(core_axis_name="c", subcore_axis_name="s")


LANES = 128


NBUF = 4
GROUPS = CHUNKS_PER_W // NBUF


@functools.partial(
    pl.kernel,
    out_type=jax.ShapeDtypeStruct((EP, LANES), jnp.float32),
    mesh=_MESH,
    scratch_types=[
        pltpu.VMEM((E_PER_W,), jnp.int32),
        pltpu.VMEM((NBUF, CHUNK, LANES), jnp.float32),
    ] + [pltpu.SemaphoreType.DMA] * (2 * NBUF),
)
def _sc_gather(h_hbm, src_hbm, u_hbm, idx_v, rows_v, *sems):
    gsems, ssems = sems[:NBUF], sems[NBUF:]
    wid = lax.axis_index("s") * NC + lax.axis_index("c")
    base = wid * E_PER_W
    pltpu.sync_copy(src_hbm.at[pl.ds(base, E_PER_W)], idx_v)

    def group(g, _):
        gh = [pltpu.async_copy(
                  h_hbm.at[idx_v.at[pl.ds((g * NBUF + b) * CHUNK, CHUNK)]],
                  rows_v.at[b], gsems[b])
              for b in range(NBUF)]
        sh = []
        for b in range(NBUF):
            gh[b].wait()
            off = base + (g * NBUF + b) * CHUNK
            sh.append(pltpu.async_copy(rows_v.at[b],
                                       u_hbm.at[pl.ds(off, CHUNK)], ssems[b]))
        for b in range(NBUF):
            sh[b].wait()
        return _

    lax.fori_loop(0, GROUPS, group, None)


@functools.partial(
    pl.kernel,
    out_type=jax.ShapeDtypeStruct((NC, NP_ROWS, MSGW), jnp.float32),
    mesh=_MESH,
    scratch_types=[
        pltpu.VMEM((CHUNKS_PER_W, CHUNK), jnp.int32),
        pltpu.VMEM((2, CHUNK, MSGW), jnp.float32),
        pltpu.VMEM_SHARED((NP_ROWS, MSGW), jnp.float32),
    ] + [pltpu.SemaphoreType.DMA] * 4,
)
def _sc_scatter(msg_hbm, dst3_hbm, zero_hbm, acc_hbm, idx_v, msg_v, shared, *sems):
    nb = 2
    lsems, scsems = sems[:nb], sems[nb:]
    cid = lax.axis_index("c")
    sid = lax.axis_index("s")
    wid = sid * NC + cid
    base = wid * E_PER_W
    row0 = sid * ROWS_PER_TILE

    pltpu.sync_copy(zero_hbm, shared.at[pl.ds(row0, ROWS_PER_TILE)])
    pltpu.sync_copy(dst3_hbm.at[wid], idx_v)
    plsc.subcore_barrier()

    def group(g, _):
        lh = [pltpu.async_copy(
                  msg_hbm.at[pl.ds(base + (g * nb + b) * CHUNK, CHUNK)],
                  msg_v.at[b], lsems[b])
              for b in range(nb)]
        sh = []
        for b in range(nb):
            lh[b].wait()
            sh.append(pltpu.async_copy(msg_v.at[b],
                                       shared.at[idx_v.at[g * nb + b]],
                                       scsems[b], add=True))
        for b in range(nb):
            sh[b].wait()
        return _

    lax.fori_loop(0, CHUNKS_PER_W // nb, group, None)
    plsc.subcore_barrier()
    pltpu.sync_copy(shared.at[pl.ds(row0, ROWS_PER_TILE)],
                    acc_hbm.at[cid, pl.ds(row0, ROWS_PER_TILE)])


def _tc_prep(x, Wp, bp):
    def body(x_ref, w_ref, b_ref, o_ref):
        h = _leaky(
            jnp.dot(x_ref[...], w_ref[...], preferred_element_type=jnp.float32)
            + b_ref[...])
        o_ref[...] = jnp.concatenate(
            [h, jnp.zeros((N_NODES, LANES - HID), jnp.float32)], axis=1)
    return pl.pallas_call(
        body, out_shape=jax.ShapeDtypeStruct((N_NODES, LANES), jnp.float32),
    )(x, Wp, bp)


def _tc_message(ea_p, u, W1, b1, W2, b2, W3r, B3, Te, Tu):
    BLK = 1024
    grid = EP // BLK

    def body(ea_ref, u_ref, w1_ref, b1_ref, w2_ref, b2_ref,
             w3_ref, b3_ref, te_ref, tu_ref, o_ref):
        e1 = _leaky(jnp.dot(ea_ref[...], w1_ref[...],
                            preferred_element_type=jnp.float32) + b1_ref[...])
        e2 = _leaky(jnp.dot(e1, w2_ref[...],
                            preferred_element_type=jnp.float32) + b2_ref[...])
        ub = u_ref[:, :HID]
        ze = jnp.dot(e2, te_ref[...], preferred_element_type=jnp.float32)
        zu = jnp.dot(ub, tu_ref[...], preferred_element_type=jnp.float32)
        z = ze * zu
        msg = (jnp.dot(z, w3_ref[...], preferred_element_type=jnp.float32)
               + jnp.dot(ub, b3_ref[...], preferred_element_type=jnp.float32))
        rows = (pl.program_id(0) * BLK
                + lax.broadcasted_iota(jnp.int32, (BLK, 1), 0))
        maskf = (rows < N_EDGES).astype(jnp.float32)
        o_ref[...] = jnp.concatenate(
            [msg * maskf, maskf, jnp.zeros((BLK, MSGW - HID - 1), jnp.float32)],
            axis=1)

    return pl.pallas_call(
        body,
        grid=(grid,),
        in_specs=[
            pl.BlockSpec((BLK, D_EDGE), lambda i: (i, 0)),
            pl.BlockSpec((BLK, LANES), lambda i: (i, 0)),
            pl.BlockSpec((D_EDGE, HID), lambda i: (0, 0)),
            pl.BlockSpec((1, HID), lambda i: (0, 0)),
            pl.BlockSpec((HID, HID), lambda i: (0, 0)),
            pl.BlockSpec((1, HID), lambda i: (0, 0)),
            pl.BlockSpec((HID * HID, HID), lambda i: (0, 0)),
            pl.BlockSpec((HID, HID), lambda i: (0, 0)),
            pl.BlockSpec((HID, HID * HID), lambda i: (0, 0)),
            pl.BlockSpec((HID, HID * HID), lambda i: (0, 0)),
        ],
        out_specs=pl.BlockSpec((BLK, MSGW), lambda i: (i, 0)),
        out_shape=jax.ShapeDtypeStruct((EP, MSGW), jnp.float32),
    )(ea_p, u, W1, b1, W2, b2, W3r, B3, Te, Tu)


def _tc_update(acc2, h128, root, root_b, ln_g, ln_b):
    def body(a_ref, h_ref, r_ref, rb_ref, g_ref, b_ref, o_ref):
        s = a_ref[0] + a_ref[1]
        deg = jnp.maximum(s[:, HID:HID + 1], 1.0)
        hb = h_ref[:, :HID]
        out = (s[:, :HID] / deg
               + jnp.dot(hb, r_ref[...], preferred_element_type=jnp.float32)
               + rb_ref[...])
        mu = jnp.mean(out, axis=-1, keepdims=True)
        var = jnp.mean((out - mu) ** 2, axis=-1, keepdims=True)
        out = (out - mu) / jnp.sqrt(var + 1e-5) * g_ref[...] + b_ref[...]
        o_ref[...] = jnp.concatenate(
            [_leaky(out) + hb, jnp.zeros((N_NODES, LANES - HID), jnp.float32)],
            axis=1)

    return pl.pallas_call(
        body, out_shape=jax.ShapeDtypeStruct((N_NODES, LANES), jnp.float32),
    )(acc2, h128, root, root_b, ln_g, ln_b)


def _tc_update_pool_head(acc2, h128, root, root_b, ln_g, ln_b,
                         batch2, W1, b1, W2, b2, W3, b3):
    def body(a_ref, h_ref, r_ref, rb_ref, g_ref, b_ref, bt_ref,
             w1_ref, b1_ref, w2_ref, b2_ref, w3_ref, b3_ref, o_ref):
        s = a_ref[0] + a_ref[1]
        deg = jnp.maximum(s[:, HID:HID + 1], 1.0)
        hb = h_ref[:, :HID]
        out = (s[:, :HID] / deg
               + jnp.dot(hb, r_ref[...], preferred_element_type=jnp.float32)
               + rb_ref[...])
        mu = jnp.mean(out, axis=-1, keepdims=True)
        var = jnp.mean((out - mu) ** 2, axis=-1, keepdims=True)
        out = (out - mu) / jnp.sqrt(var + 1e-5) * g_ref[...] + b_ref[...]
        hn = _leaky(out) + hb
        gids = lax.broadcasted_iota(jnp.int32, (N_GRAPHS, N_NODES), 0)
        mask = (bt_ref[...] == gids).astype(jnp.float32)
        cnt = jnp.maximum(jnp.sum(mask, axis=1, keepdims=True), 1.0)
        pooled = jnp.dot(mask, hn, preferred_element_type=jnp.float32) / cnt
        z = _leaky(jnp.dot(pooled, w1_ref[...],
                           preferred_element_type=jnp.float32) + b1_ref[...])
        z = _leaky(jnp.dot(z, w2_ref[...],
                           preferred_element_type=jnp.float32) + b2_ref[...])
        o_ref[...] = jnp.dot(z, w3_ref[...],
                             preferred_element_type=jnp.float32) + b3_ref[...]

    return pl.pallas_call(
        body, out_shape=jax.ShapeDtypeStruct((N_GRAPHS, 1), jnp.float32),
    )(acc2, h128, root, root_b, ln_g, ln_b, batch2, W1, b1, W2, b2, W3, b3)


def kernel(x, edge_index, edge_attr, batch, params):
    src = edge_index[0]
    dst = edge_index[1]
    src_p = jnp.pad(src, (0, EP - N_EDGES))
    dst3 = jnp.pad(dst, (0, EP - N_EDGES)).reshape(NW, CHUNKS_PER_W, CHUNK)
    ea_p = jnp.pad(edge_attr, ((0, EP - N_EDGES), (0, 0)))
    batch2 = batch.reshape(1, N_NODES)
    zero_rows = jnp.zeros((ROWS_PER_TILE, MSGW), jnp.float32)
    eye = jnp.eye(HID, dtype=jnp.float32)
    Te = jnp.repeat(eye, HID, axis=1)
    Tu = jnp.tile(eye, (1, HID))

    def r2(v):
        return v.reshape(1, -1)

    def msg_args(blk):
        return (blk["W1"], r2(blk["b1"]), blk["W2"], r2(blk["b2"]),
                blk["W3"].reshape(HID * HID, HID),
                blk["b3"].reshape(HID, HID), Te, Tu)

    hd = params["head"]
    blk0, blk1 = params["blocks"]
    h = _tc_prep(x, params["Wp"], r2(params["bp"]))
    u = _sc_gather(h, src_p)
    msgp = _tc_message(ea_p, u, *msg_args(blk0))
    acc2 = _sc_scatter(msgp, dst3, zero_rows)
    h = _tc_update(acc2[:, :N_NODES, :], h, blk0["root"], r2(blk0["root_b"]),
                   r2(blk0["ln_g"]), r2(blk0["ln_b"]))
    u = _sc_gather(h, src_p)
    msgp = _tc_message(ea_p, u, *msg_args(blk1))
    acc2 = _sc_scatter(msgp, dst3, zero_rows)
    pred = _tc_update_pool_head(
        acc2[:, :N_NODES, :], h, blk1["root"], r2(blk1["root_b"]),
        r2(blk1["ln_g"]), r2(blk1["ln_b"]), batch2,
        hd["W1"], r2(hd["b1"]), hd["W2"], r2(hd["b2"]), hd["W3"], r2(hd["b3"]))
    return pred.reshape(N_GRAPHS)


# speedup vs baseline: 2.2173x; 1.0168x over previous
import functools

import jax
import jax.numpy as jnp
from jax import lax
from jax.experimental import pallas as pl
from jax.experimental.pallas import tpu as pltpu, tpu_sc as plsc

N_NODES = 10000
N_EDGES = 160000
N_GRAPHS = 16
D_IN = 4
D_EDGE = 5
HID = 32

NC = 2
NS = 16
NW = NC * NS
CHUNK = 128
CHUNKS_PER_W = 40
EP = NW * CHUNKS_PER_W * CHUNK
E_PER_W = CHUNKS_PER_W * CHUNK
MSGW = 128
NP_ROWS = 10240
ROWS_PER_TILE = NP_ROWS // NS


def _leaky(v):
    return jnp.where(v >= 0, v, 0.1 * v)


_MESH = plsc.VectorSubcoreMesh(core_axis_name="c", subcore_axis_name="s")


LANES = 128


NBUF = 5
GROUPS = CHUNKS_PER_W // NBUF


@functools.partial(
    pl.kernel,
    out_type=jax.ShapeDtypeStruct((EP, LANES), jnp.float32),
    mesh=_MESH,
    scratch_types=[
        pltpu.VMEM((E_PER_W,), jnp.int32),
        pltpu.VMEM((NBUF, CHUNK, LANES), jnp.float32),
    ] + [pltpu.SemaphoreType.DMA] * (2 * NBUF),
)
def _sc_gather(h_hbm, src_hbm, u_hbm, idx_v, rows_v, *sems):
    gsems, ssems = sems[:NBUF], sems[NBUF:]
    wid = lax.axis_index("s") * NC + lax.axis_index("c")
    base = wid * E_PER_W
    pltpu.sync_copy(src_hbm.at[pl.ds(base, E_PER_W)], idx_v)

    def group(g, _):
        gh = [pltpu.async_copy(
                  h_hbm.at[idx_v.at[pl.ds((g * NBUF + b) * CHUNK, CHUNK)]],
                  rows_v.at[b], gsems[b])
              for b in range(NBUF)]
        sh = []
        for b in range(NBUF):
            gh[b].wait()
            off = base + (g * NBUF + b) * CHUNK
            sh.append(pltpu.async_copy(rows_v.at[b],
                                       u_hbm.at[pl.ds(off, CHUNK)], ssems[b]))
        for b in range(NBUF):
            sh[b].wait()
        return _

    lax.fori_loop(0, GROUPS, group, None)


@functools.partial(
    pl.kernel,
    out_type=jax.ShapeDtypeStruct((NC, NP_ROWS, MSGW), jnp.float32),
    mesh=_MESH,
    scratch_types=[
        pltpu.VMEM((CHUNKS_PER_W, CHUNK), jnp.int32),
        pltpu.VMEM((2, CHUNK, MSGW), jnp.float32),
        pltpu.VMEM_SHARED((NP_ROWS, MSGW), jnp.float32),
    ] + [pltpu.SemaphoreType.DMA] * 4,
)
def _sc_scatter(msg_hbm, dst3_hbm, zero_hbm, acc_hbm, idx_v, msg_v, shared, *sems):
    nb = 2
    lsems, scsems = sems[:nb], sems[nb:]
    cid = lax.axis_index("c")
    sid = lax.axis_index("s")
    wid = sid * NC + cid
    base = wid * E_PER_W
    row0 = sid * ROWS_PER_TILE

    pltpu.sync_copy(zero_hbm, shared.at[pl.ds(row0, ROWS_PER_TILE)])
    pltpu.sync_copy(dst3_hbm.at[wid], idx_v)
    plsc.subcore_barrier()

    def group(g, _):
        lh = [pltpu.async_copy(
                  msg_hbm.at[pl.ds(base + (g * nb + b) * CHUNK, CHUNK)],
                  msg_v.at[b], lsems[b])
              for b in range(nb)]
        sh = []
        for b in range(nb):
            lh[b].wait()
            sh.append(pltpu.async_copy(msg_v.at[b],
                                       shared.at[idx_v.at[g * nb + b]],
                                       scsems[b], add=True))
        for b in range(nb):
            sh[b].wait()
        return _

    lax.fori_loop(0, CHUNKS_PER_W // nb, group, None)
    plsc.subcore_barrier()
    pltpu.sync_copy(shared.at[pl.ds(row0, ROWS_PER_TILE)],
                    acc_hbm.at[cid, pl.ds(row0, ROWS_PER_TILE)])


def _tc_prep(x, Wp, bp):
    def body(x_ref, w_ref, b_ref, o_ref):
        h = _leaky(
            jnp.dot(x_ref[...], w_ref[...], preferred_element_type=jnp.float32)
            + b_ref[...])
        o_ref[...] = jnp.concatenate(
            [h, jnp.zeros((N_NODES, LANES - HID), jnp.float32)], axis=1)
    return pl.pallas_call(
        body, out_shape=jax.ShapeDtypeStruct((N_NODES, LANES), jnp.float32),
    )(x, Wp, bp)


def _tc_message(ea_p, u, W1, b1, W2, b2, W3r, B3, Te, Tu):
    BLK = 1024
    grid = EP // BLK

    def body(ea_ref, u_ref, w1_ref, b1_ref, w2_ref, b2_ref,
             w3_ref, b3_ref, te_ref, tu_ref, o_ref):
        e1 = _leaky(jnp.dot(ea_ref[...], w1_ref[...],
                            preferred_element_type=jnp.float32) + b1_ref[...])
        e2 = _leaky(jnp.dot(e1, w2_ref[...],
                            preferred_element_type=jnp.float32) + b2_ref[...])
        ub = u_ref[:, :HID]
        ze = jnp.dot(e2, te_ref[...], preferred_element_type=jnp.float32)
        zu = jnp.dot(ub, tu_ref[...], preferred_element_type=jnp.float32)
        z = ze * zu
        msg = (jnp.dot(z, w3_ref[...], preferred_element_type=jnp.float32)
               + jnp.dot(ub, b3_ref[...], preferred_element_type=jnp.float32))
        rows = (pl.program_id(0) * BLK
                + lax.broadcasted_iota(jnp.int32, (BLK, 1), 0))
        maskf = (rows < N_EDGES).astype(jnp.float32)
        o_ref[...] = jnp.concatenate(
            [msg * maskf, maskf, jnp.zeros((BLK, MSGW - HID - 1), jnp.float32)],
            axis=1)

    return pl.pallas_call(
        body,
        grid=(grid,),
        in_specs=[
            pl.BlockSpec((BLK, D_EDGE), lambda i: (i, 0)),
            pl.BlockSpec((BLK, LANES), lambda i: (i, 0)),
            pl.BlockSpec((D_EDGE, HID), lambda i: (0, 0)),
            pl.BlockSpec((1, HID), lambda i: (0, 0)),
            pl.BlockSpec((HID, HID), lambda i: (0, 0)),
            pl.BlockSpec((1, HID), lambda i: (0, 0)),
            pl.BlockSpec((HID * HID, HID), lambda i: (0, 0)),
            pl.BlockSpec((HID, HID), lambda i: (0, 0)),
            pl.BlockSpec((HID, HID * HID), lambda i: (0, 0)),
            pl.BlockSpec((HID, HID * HID), lambda i: (0, 0)),
        ],
        out_specs=pl.BlockSpec((BLK, MSGW), lambda i: (i, 0)),
        out_shape=jax.ShapeDtypeStruct((EP, MSGW), jnp.float32),
    )(ea_p, u, W1, b1, W2, b2, W3r, B3, Te, Tu)


def _tc_update(acc2, h128, root, root_b, ln_g, ln_b):
    def body(a_ref, h_ref, r_ref, rb_ref, g_ref, b_ref, o_ref):
        s = a_ref[0] + a_ref[1]
        deg = jnp.maximum(s[:, HID:HID + 1], 1.0)
        hb = h_ref[:, :HID]
        out = (s[:, :HID] / deg
               + jnp.dot(hb, r_ref[...], preferred_element_type=jnp.float32)
               + rb_ref[...])
        mu = jnp.mean(out, axis=-1, keepdims=True)
        var = jnp.mean((out - mu) ** 2, axis=-1, keepdims=True)
        out = (out - mu) / jnp.sqrt(var + 1e-5) * g_ref[...] + b_ref[...]
        o_ref[...] = jnp.concatenate(
            [_leaky(out) + hb, jnp.zeros((N_NODES, LANES - HID), jnp.float32)],
            axis=1)

    return pl.pallas_call(
        body, out_shape=jax.ShapeDtypeStruct((N_NODES, LANES), jnp.float32),
    )(acc2, h128, root, root_b, ln_g, ln_b)


def _tc_update_pool_head(acc2, h128, root, root_b, ln_g, ln_b,
                         batch2, W1, b1, W2, b2, W3, b3):
    def body(a_ref, h_ref, r_ref, rb_ref, g_ref, b_ref, bt_ref,
             w1_ref, b1_ref, w2_ref, b2_ref, w3_ref, b3_ref, o_ref):
        s = a_ref[0] + a_ref[1]
        deg = jnp.maximum(s[:, HID:HID + 1], 1.0)
        hb = h_ref[:, :HID]
        out = (s[:, :HID] / deg
               + jnp.dot(hb, r_ref[...], preferred_element_type=jnp.float32)
               + rb_ref[...])
        mu = jnp.mean(out, axis=-1, keepdims=True)
        var = jnp.mean((out - mu) ** 2, axis=-1, keepdims=True)
        out = (out - mu) / jnp.sqrt(var + 1e-5) * g_ref[...] + b_ref[...]
        hn = _leaky(out) + hb
        gids = lax.broadcasted_iota(jnp.int32, (N_GRAPHS, N_NODES), 0)
        mask = (bt_ref[...] == gids).astype(jnp.float32)
        cnt = jnp.maximum(jnp.sum(mask, axis=1, keepdims=True), 1.0)
        pooled = jnp.dot(mask, hn, preferred_element_type=jnp.float32) / cnt
        z = _leaky(jnp.dot(pooled, w1_ref[...],
                           preferred_element_type=jnp.float32) + b1_ref[...])
        z = _leaky(jnp.dot(z, w2_ref[...],
                           preferred_element_type=jnp.float32) + b2_ref[...])
        o_ref[...] = jnp.dot(z, w3_ref[...],
                             preferred_element_type=jnp.float32) + b3_ref[...]

    return pl.pallas_call(
        body, out_shape=jax.ShapeDtypeStruct((N_GRAPHS, 1), jnp.float32),
    )(acc2, h128, root, root_b, ln_g, ln_b, batch2, W1, b1, W2, b2, W3, b3)


def kernel(x, edge_index, edge_attr, batch, params):
    src = edge_index[0]
    dst = edge_index[1]
    src_p = jnp.pad(src, (0, EP - N_EDGES))
    dst3 = jnp.pad(dst, (0, EP - N_EDGES)).reshape(NW, CHUNKS_PER_W, CHUNK)
    ea_p = jnp.pad(edge_attr, ((0, EP - N_EDGES), (0, 0)))
    batch2 = batch.reshape(1, N_NODES)
    zero_rows = jnp.zeros((ROWS_PER_TILE, MSGW), jnp.float32)
    eye = jnp.eye(HID, dtype=jnp.float32)
    Te = jnp.repeat(eye, HID, axis=1)
    Tu = jnp.tile(eye, (1, HID))

    def r2(v):
        return v.reshape(1, -1)

    def msg_args(blk):
        return (blk["W1"], r2(blk["b1"]), blk["W2"], r2(blk["b2"]),
                blk["W3"].reshape(HID * HID, HID),
                blk["b3"].reshape(HID, HID), Te, Tu)

    hd = params["head"]
    blk0, blk1 = params["blocks"]
    h = _tc_prep(x, params["Wp"], r2(params["bp"]))
    u = _sc_gather(h, src_p)
    msgp = _tc_message(ea_p, u, *msg_args(blk0))
    acc2 = _sc_scatter(msgp, dst3, zero_rows)
    h = _tc_update(acc2[:, :N_NODES, :], h, blk0["root"], r2(blk0["root_b"]),
                   r2(blk0["ln_g"]), r2(blk0["ln_b"]))
    u = _sc_gather(h, src_p)
    msgp = _tc_message(ea_p, u, *msg_args(blk1))
    acc2 = _sc_scatter(msgp, dst3, zero_rows)
    pred = _tc_update_pool_head(
        acc2[:, :N_NODES, :], h, blk1["root"], r2(blk1["root_b"]),
        r2(blk1["ln_g"]), r2(blk1["ln_b"]), batch2,
        hd["W1"], r2(hd["b1"]), hd["W2"], r2(hd["b2"]), hd["W3"], r2(hd["b3"]))
    return pred.reshape(N_GRAPHS)
